# baseline (speedup 1.0000x reference)
import functools

import jax
import jax.numpy as jnp
from jax import lax
from jax.experimental import pallas as pl
from jax.experimental.pallas import tpu as pltpu
from jax.experimental.pallas import tpu_sc as plsc

B = 16384
D_ID = 128
BB = 2048
K_SMALL = 264


def _make_sc_gather(V, D, batch):
    info = plsc.get_sparse_core_info()
    NC, NS = info.num_cores, info.num_subcores
    NW = NC * NS
    b_per_w = batch // NW
    mesh = plsc.VectorSubcoreMesh(core_axis_name="c", subcore_axis_name="s")

    nch = 4
    C = b_per_w // nch

    @functools.partial(
        pl.kernel,
        mesh=mesh,
        out_type=jax.ShapeDtypeStruct((batch, D), jnp.float32),
        scratch_types=[
            pltpu.VMEM((b_per_w,), jnp.int32),
            pltpu.VMEM((b_per_w, D), jnp.float32),
            pltpu.SemaphoreType.DMA,
            pltpu.SemaphoreType.DMA,
            pltpu.SemaphoreType.DMA,
        ],
    )
    def sc_gather(table_hbm, idx_hbm, out_hbm, idx_v, rows_v, gs0, gs1, ws):
        wid = lax.axis_index("s") * NC + lax.axis_index("c")
        base = wid * b_per_w
        pltpu.sync_copy(idx_hbm.at[pl.ds(base, b_per_w)], idx_v)
        gsems = (gs0, gs1)
        gh = [
            pltpu.async_copy(table_hbm.at[idx_v.at[pl.ds(c * C, C)]],
                             rows_v.at[pl.ds(c * C, C)], gsems[c % 2])
            for c in range(nch)
        ]
        wh = []
        for c in range(nch):
            gh[c].wait()
            wh.append(pltpu.async_copy(
                rows_v.at[pl.ds(c * C, C)],
                out_hbm.at[pl.ds(base + c * C, C)], ws))
        for h in wh:
            h.wait()

    return sc_gather


def _mlp_body(cidx_ref, rows_ref, gt_ref, ut_ref, tt_ref, st_ref,
              w1_ref, b1_ref, w2_ref, b2_ref, out_ref, mall_ref):
    f32 = jnp.float32

    @pl.when(pl.program_id(0) == 0)
    def _fold():
        mall_ref[0:16, :] = (jnp.dot(gt_ref[...], w1_ref[0:32, :],
                                     preferred_element_type=f32) * 2.0
                             + b1_ref[...])
        mall_ref[16:40, :] = jnp.dot(ut_ref[...], w1_ref[32:64, :],
                                     preferred_element_type=f32)
        mall_ref[40:248, :] = jnp.dot(tt_ref[...], w1_ref[64:96, :],
                                      preferred_element_type=f32)
        mall_ref[248:264, :] = jnp.dot(st_ref[...], w1_ref[96:128, :],
                                       preferred_element_type=f32)

    iota = lax.broadcasted_iota(jnp.int32, (BB, K_SMALL), 1)
    sel = jnp.where(iota < 16, cidx_ref[:, 0:1],
                    jnp.where(iota < 40, cidx_ref[:, 1:2],
                              jnp.where(iota < 248, cidx_ref[:, 2:3],
                                        cidx_ref[:, 3:4])))
    mh = (iota == sel).astype(f32)
    x1t = (lax.dot_general(mall_ref[...], mh, (((0,), (1,)), ((), ())),
                           preferred_element_type=f32)
           + lax.dot_general(w1_ref[128:256, :], rows_ref[...],
                             (((0,), (1,)), ((), ())),
                             preferred_element_type=f32))
    ht = jnp.maximum(x1t, 0.0)
    out_ref[...] = lax.dot_general(w2_ref[...], ht, (((0,), (0,)), ((), ())),
                                   preferred_element_type=f32) + b2_ref[...]


def _pad_rows(t, rows):
    return jnp.pad(t, ((0, rows - t.shape[0]), (0, 0)))


def kernel(gender_idx, usage_idx, articleType_idx, season_idx, id_idx,
           gender_table, usage_table, type_table, season_table, id_table,
           W1, b1, W2, b2):
    i32 = jnp.int32
    cidx = jnp.stack([
        gender_idx.astype(i32),
        usage_idx.astype(i32) + 16,
        articleType_idx.astype(i32) + 40,
        season_idx.astype(i32) + 248,
    ], axis=-1)
    ids = id_idx.astype(i32)

    rows = _make_sc_gather(id_table.shape[0], D_ID, B)(id_table, ids)

    gt = _pad_rows(gender_table, 16)
    ut = _pad_rows(usage_table, 24)
    tt = _pad_rows(type_table, 208)
    st = _pad_rows(season_table, 16)

    grid = (B // BB,)
    full = lambda shape: pl.BlockSpec(shape, lambda i: (0,) * len(shape))
    out = pl.pallas_call(
        _mlp_body,
        grid=grid,
        in_specs=[
            pl.BlockSpec((BB, 4), lambda i: (i, 0)),
            pl.BlockSpec((BB, D_ID), lambda i: (i, 0)),
            full((16, 32)), full((24, 32)), full((208, 32)), full((16, 32)),
            full((256, 64)), full((1, 64)), full((64, 32)), full((32, 1)),
        ],
        out_specs=pl.BlockSpec((32, BB), lambda i: (0, i)),
        out_shape=jax.ShapeDtypeStruct((32, B), jnp.float32),
        scratch_shapes=[pltpu.VMEM((K_SMALL, 64), jnp.float32)],
    )(cidx, rows, gt, ut, tt, st,
      W1.astype(jnp.float32), b1.reshape(1, 64), W2, b2.reshape(32, 1))
    return out.T


# speedup vs baseline: 1.0184x; 1.0184x over previous
import functools

import jax
import jax.numpy as jnp
from jax import lax
from jax.experimental import pallas as pl
from jax.experimental.pallas import tpu as pltpu
from jax.experimental.pallas import tpu_sc as plsc

B = 16384
D_ID = 128
BB = 4096
K_SMALL = 264


def _make_sc_gather(V, D, batch):
    info = plsc.get_sparse_core_info()
    NC, NS = info.num_cores, info.num_subcores
    NW = NC * NS
    b_per_w = batch // NW
    mesh = plsc.VectorSubcoreMesh(core_axis_name="c", subcore_axis_name="s")

    nch = 4
    C = b_per_w // nch

    @functools.partial(
        pl.kernel,
        mesh=mesh,
        out_type=jax.ShapeDtypeStruct((batch, D), jnp.float32),
        scratch_types=[
            pltpu.VMEM((b_per_w,), jnp.int32),
            pltpu.VMEM((b_per_w, D), jnp.float32),
            pltpu.SemaphoreType.DMA,
            pltpu.SemaphoreType.DMA,
            pltpu.SemaphoreType.DMA,
        ],
    )
    def sc_gather(table_hbm, idx_hbm, out_hbm, idx_v, rows_v, gs0, gs1, ws):
        wid = lax.axis_index("s") * NC + lax.axis_index("c")
        base = wid * b_per_w
        pltpu.sync_copy(idx_hbm.at[pl.ds(base, b_per_w)], idx_v)
        gsems = (gs0, gs1)
        gh = [
            pltpu.async_copy(table_hbm.at[idx_v.at[pl.ds(c * C, C)]],
                             rows_v.at[pl.ds(c * C, C)], gsems[c % 2])
            for c in range(nch)
        ]
        wh = []
        for c in range(nch):
            gh[c].wait()
            wh.append(pltpu.async_copy(
                rows_v.at[pl.ds(c * C, C)],
                out_hbm.at[pl.ds(base + c * C, C)], ws))
        for h in wh:
            h.wait()

    return sc_gather


def _mlp_body(cidx_ref, rows_ref, gt_ref, ut_ref, tt_ref, st_ref,
              w1_ref, b1_ref, w2_ref, b2_ref, out_ref, mall_ref):
    f32 = jnp.float32

    @pl.when(pl.program_id(0) == 0)
    def _fold():
        mall_ref[0:16, :] = (jnp.dot(gt_ref[...], w1_ref[0:32, :],
                                     preferred_element_type=f32) * 2.0
                             + b1_ref[...])
        mall_ref[16:40, :] = jnp.dot(ut_ref[...], w1_ref[32:64, :],
                                     preferred_element_type=f32)
        mall_ref[40:248, :] = jnp.dot(tt_ref[...], w1_ref[64:96, :],
                                      preferred_element_type=f32)
        mall_ref[248:264, :] = jnp.dot(st_ref[...], w1_ref[96:128, :],
                                       preferred_element_type=f32)

    iota = lax.broadcasted_iota(jnp.int32, (BB, K_SMALL), 1)
    sel = jnp.where(iota < 16, cidx_ref[:, 0:1],
                    jnp.where(iota < 40, cidx_ref[:, 1:2],
                              jnp.where(iota < 248, cidx_ref[:, 2:3],
                                        cidx_ref[:, 3:4])))
    mh = (iota == sel).astype(f32)
    x1t = (lax.dot_general(mall_ref[...], mh, (((0,), (1,)), ((), ())),
                           preferred_element_type=f32)
           + lax.dot_general(w1_ref[128:256, :], rows_ref[...],
                             (((0,), (1,)), ((), ())),
                             preferred_element_type=f32))
    ht = jnp.maximum(x1t, 0.0)
    out_ref[...] = lax.dot_general(w2_ref[...], ht, (((0,), (0,)), ((), ())),
                                   preferred_element_type=f32) + b2_ref[...]


def _pad_rows(t, rows):
    return jnp.pad(t, ((0, rows - t.shape[0]), (0, 0)))


def kernel(gender_idx, usage_idx, articleType_idx, season_idx, id_idx,
           gender_table, usage_table, type_table, season_table, id_table,
           W1, b1, W2, b2):
    i32 = jnp.int32
    cidx = jnp.stack([
        gender_idx.astype(i32),
        usage_idx.astype(i32) + 16,
        articleType_idx.astype(i32) + 40,
        season_idx.astype(i32) + 248,
    ], axis=-1)
    ids = id_idx.astype(i32)

    rows = _make_sc_gather(id_table.shape[0], D_ID, B)(id_table, ids)

    gt = _pad_rows(gender_table, 16)
    ut = _pad_rows(usage_table, 24)
    tt = _pad_rows(type_table, 208)
    st = _pad_rows(season_table, 16)

    grid = (B // BB,)
    full = lambda shape: pl.BlockSpec(shape, lambda i: (0,) * len(shape))
    out = pl.pallas_call(
        _mlp_body,
        grid=grid,
        in_specs=[
            pl.BlockSpec((BB, 4), lambda i: (i, 0)),
            pl.BlockSpec((BB, D_ID), lambda i: (i, 0)),
            full((16, 32)), full((24, 32)), full((208, 32)), full((16, 32)),
            full((256, 64)), full((1, 64)), full((64, 32)), full((32, 1)),
        ],
        out_specs=pl.BlockSpec((32, BB), lambda i: (0, i)),
        out_shape=jax.ShapeDtypeStruct((32, B), jnp.float32),
        scratch_shapes=[pltpu.VMEM((K_SMALL, 64), jnp.float32)],
    )(cidx, rows, gt, ut, tt, st,
      W1.astype(jnp.float32), b1.reshape(1, 64), W2, b2.reshape(32, 1))
    return out.T
